# baseline (speedup 1.0000x reference)
import functools

import jax
import jax.numpy as jnp
from jax.experimental import pallas as pl
from jax.experimental.pallas import tpu as pltpu

_EPS = 1e-5


def _mm_stats_body(x_ref, w_ref, b_ref, h_ref, st_ref):
    i = pl.program_id(0)
    h = jnp.dot(x_ref[...], w_ref[...], preferred_element_type=jnp.float32)
    h = h + b_ref[...]
    h_ref[...] = h
    s = jnp.sum(h, axis=0, keepdims=True)
    sq = jnp.sum(h * h, axis=0, keepdims=True)
    part = jnp.concatenate([s, sq], axis=0)

    @pl.when(i == 0)
    def _():
        st_ref[...] = part

    @pl.when(i != 0)
    def _():
        st_ref[...] = st_ref[...] + part


def _bn_mm_stats_body(n_rows, h_ref, st_in_ref, g_ref, be_ref, w_ref, b_ref,
                      o_ref, st_out_ref):
    i = pl.program_id(0)
    st = st_in_ref[...]
    inv_n = 1.0 / n_rows
    mean = st[0:1, :] * inv_n
    var = st[1:2, :] * inv_n - mean * mean
    scale = jax.lax.rsqrt(var + _EPS) * g_ref[...]
    shift = be_ref[...] - mean * scale
    a = jnp.maximum(h_ref[...] * scale + shift, 0.0)
    h2 = jnp.dot(a, w_ref[...], preferred_element_type=jnp.float32)
    h2 = h2 + b_ref[...]
    o_ref[...] = h2
    s = jnp.sum(h2, axis=0, keepdims=True)
    sq = jnp.sum(h2 * h2, axis=0, keepdims=True)
    part = jnp.concatenate([s, sq], axis=0)

    @pl.when(i == 0)
    def _():
        st_out_ref[...] = part

    @pl.when(i != 0)
    def _():
        st_out_ref[...] = st_out_ref[...] + part


def _bn_mm_body(n_rows, h_ref, st_in_ref, g_ref, be_ref, w_ref, b_ref, o_ref):
    st = st_in_ref[...]
    inv_n = 1.0 / n_rows
    mean = st[0:1, :] * inv_n
    var = st[1:2, :] * inv_n - mean * mean
    scale = jax.lax.rsqrt(var + _EPS) * g_ref[...]
    shift = be_ref[...] - mean * scale
    a = jnp.maximum(h_ref[...] * scale + shift, 0.0)
    h2 = jnp.dot(a, w_ref[...], preferred_element_type=jnp.float32)
    o_ref[...] = h2 + b_ref[...]


def _row_spec(blk, d):
    return pl.BlockSpec((blk, d), lambda i: (i, 0))


def _full_spec(shape):
    nd = len(shape)
    return pl.BlockSpec(shape, lambda i: (0,) * nd)


def kernel(input, W1, b1, g1, be1, W2, b2, g2, be2, W3, b3):
    n, d = input.shape
    f = W1.shape[0]
    blk = 2000
    grid = (n // blk,)

    w1t = W1.T
    w2t = W2.T
    w3t = W3.T
    b1r = b1.reshape(1, f)
    b2r = b2.reshape(1, f)
    b3r = b3.reshape(1, f)
    g1r = g1.reshape(1, f)
    g2r = g2.reshape(1, f)
    be1r = be1.reshape(1, f)
    be2r = be2.reshape(1, f)

    h1, st1 = pl.pallas_call(
        _mm_stats_body,
        grid=grid,
        in_specs=[_row_spec(blk, d), _full_spec((d, f)), _full_spec((1, f))],
        out_specs=[_row_spec(blk, f), _full_spec((2, f))],
        out_shape=[
            jax.ShapeDtypeStruct((n, f), jnp.float32),
            jax.ShapeDtypeStruct((2, f), jnp.float32),
        ],
        compiler_params=pltpu.CompilerParams(
            dimension_semantics=("arbitrary",)),
    )(input, w1t, b1r)

    h2, st2 = pl.pallas_call(
        functools.partial(_bn_mm_stats_body, float(n)),
        grid=grid,
        in_specs=[_row_spec(blk, f), _full_spec((2, f)), _full_spec((1, f)),
                  _full_spec((1, f)), _full_spec((f, f)), _full_spec((1, f))],
        out_specs=[_row_spec(blk, f), _full_spec((2, f))],
        out_shape=[
            jax.ShapeDtypeStruct((n, f), jnp.float32),
            jax.ShapeDtypeStruct((2, f), jnp.float32),
        ],
        compiler_params=pltpu.CompilerParams(
            dimension_semantics=("arbitrary",)),
    )(h1, st1, g1r, be1r, w2t, b2r)

    out = pl.pallas_call(
        functools.partial(_bn_mm_body, float(n)),
        grid=grid,
        in_specs=[_row_spec(blk, f), _full_spec((2, f)), _full_spec((1, f)),
                  _full_spec((1, f)), _full_spec((f, f)), _full_spec((1, f))],
        out_specs=_row_spec(blk, f),
        out_shape=jax.ShapeDtypeStruct((n, f), jnp.float32),
        compiler_params=pltpu.CompilerParams(
            dimension_semantics=("arbitrary",)),
    )(h2, st2, g2r, be2r, w3t, b3r)

    return out
